# Initial kernel scaffold; baseline (speedup 1.0000x reference)
import jax
import jax.numpy as jnp
from jax.experimental import pallas as pl


def kernel(x, edge_index, Wt1, bt1, Wp1, bp1, Wt2, bt2, Wp2, bp2, Wt3, bt3, Wp3, bp3, Wl, bl, Wl2, bl2, Wo, bo):
    raise NotImplementedError("write your pallas kernel here")


# speedup vs baseline: 1.9360x; 1.9360x over previous
import functools

import jax
import jax.numpy as jnp
from jax.experimental import pallas as pl

N = 10000
HID = 128
BLK = 1000


def _mm_kernel(h_ref, w_ref, b_ref, o_ref):
    o_ref[...] = (
        jnp.dot(h_ref[...], w_ref[...], preferred_element_type=jnp.float32)
        + b_ref[...]
    )


def _dense(h, W, b):
    n, k = h.shape
    m = W.shape[1]
    return pl.pallas_call(
        _mm_kernel,
        grid=(n // BLK,),
        in_specs=[
            pl.BlockSpec((BLK, k), lambda i: (i, 0)),
            pl.BlockSpec((k, m), lambda i: (0, 0)),
            pl.BlockSpec((1, m), lambda i: (0, 0)),
        ],
        out_specs=pl.BlockSpec((BLK, m), lambda i: (i, 0)),
        out_shape=jax.ShapeDtypeStruct((n, m), jnp.float32),
    )(h, W, b.reshape(1, -1))


def _head_kernel(h_ref, wl_ref, bl_ref, wl2_ref, bl2_ref, o_ref):
    i = pl.program_id(0)
    t = jnp.dot(h_ref[...], wl_ref[...], preferred_element_type=jnp.float32)
    t = jnp.maximum(t + bl_ref[...], 0.0)
    t = jnp.dot(t, wl2_ref[...], preferred_element_type=jnp.float32)
    t = jnp.maximum(t + bl2_ref[...], 0.0)
    s = jnp.sum(t, axis=0, keepdims=True)

    @pl.when(i == 0)
    def _():
        o_ref[...] = jnp.zeros_like(o_ref)

    o_ref[...] += s


def _head(h, Wl, bl, Wl2, bl2):
    m1 = Wl.shape[1]
    m2 = Wl2.shape[1]
    return pl.pallas_call(
        _head_kernel,
        grid=(h.shape[0] // BLK,),
        in_specs=[
            pl.BlockSpec((BLK, h.shape[1]), lambda i: (i, 0)),
            pl.BlockSpec((h.shape[1], m1), lambda i: (0, 0)),
            pl.BlockSpec((1, m1), lambda i: (0, 0)),
            pl.BlockSpec((m1, m2), lambda i: (0, 0)),
            pl.BlockSpec((1, m2), lambda i: (0, 0)),
        ],
        out_specs=pl.BlockSpec((1, m2), lambda i: (0, 0)),
        out_shape=jax.ShapeDtypeStruct((1, m2), jnp.float32),
    )(h, Wl, bl.reshape(1, -1), Wl2, bl2.reshape(1, -1))


def kernel(x, edge_index, Wt1, bt1, Wp1, bp1, Wt2, bt2, Wp2, bp2, Wt3, bt3,
           Wp3, bp3, Wl, bl, Wl2, bl2, Wo, bo):
    src = edge_index[0]
    dst = edge_index[1]
    h = x
    for Wt, bt, Wp, bp in ((Wt1, bt1, Wp1, bp1), (Wt2, bt2, Wp2, bp2),
                           (Wt3, bt3, Wp3, bp3)):
        W = jnp.concatenate([Wt + Wp, Wt], axis=1)
        b = jnp.concatenate([bt + bp, jnp.zeros_like(bt)])
        AB = _dense(h, W, b)
        A = AB[:, :HID]
        B = AB[:, HID:]
        C = jax.ops.segment_min(B[src], dst, num_segments=N)
        h = A - C
        h = jnp.where(jnp.isfinite(h), h, 0.0)
    s = _head(h, Wl, bl, Wl2, bl2)
    out = (s / N) @ Wo + bo
    return out.reshape(-1)


# speedup vs baseline: 2.3128x; 1.1947x over previous
import functools

import jax
import jax.numpy as jnp
from jax import lax
from jax.experimental import pallas as pl
from jax.experimental.pallas import tpu as pltpu
from jax.experimental.pallas import tpu_sc as plsc

N = 10000
E = 320000
HID = 128
BLK = 1000

NW = 32
RPW = 313
NPAD = NW * RPW
CH = 2000
FLUSH = 8192
CAPW = 40 * FLUSH
GB = 128

_mesh = plsc.VectorSubcoreMesh(core_axis_name="c", subcore_axis_name="s")
_sc_params = pltpu.CompilerParams(needs_layout_passes=False)


def _wid():
    return lax.axis_index("s") * 2 + lax.axis_index("c")


def _mm_kernel(h_ref, w_ref, b_ref, o_ref):
    o_ref[...] = (
        jnp.dot(h_ref[...], w_ref[...], preferred_element_type=jnp.float32)
        + b_ref[...]
    )


def _dense(h, W, b):
    n, k = h.shape
    m = W.shape[1]
    return pl.pallas_call(
        _mm_kernel,
        grid=(n // BLK,),
        in_specs=[
            pl.BlockSpec((BLK, k), lambda i: (i, 0)),
            pl.BlockSpec((k, m), lambda i: (0, 0)),
            pl.BlockSpec((1, m), lambda i: (0, 0)),
        ],
        out_specs=pl.BlockSpec((BLK, m), lambda i: (i, 0)),
        out_shape=jax.ShapeDtypeStruct((n, m), jnp.float32),
    )(h, W, b.reshape(1, -1))


def _head_kernel(h_ref, wl_ref, bl_ref, wl2_ref, bl2_ref, o_ref):
    i = pl.program_id(0)
    t = jnp.dot(h_ref[...], wl_ref[...], preferred_element_type=jnp.float32)
    t = jnp.maximum(t + bl_ref[...], 0.0)
    t = jnp.dot(t, wl2_ref[...], preferred_element_type=jnp.float32)
    t = jnp.maximum(t + bl2_ref[...], 0.0)
    s = jnp.sum(t, axis=0, keepdims=True)

    @pl.when(i == 0)
    def _():
        o_ref[...] = jnp.zeros_like(o_ref)

    o_ref[...] += s


def _head(h, Wl, bl, Wl2, bl2):
    m1 = Wl.shape[1]
    m2 = Wl2.shape[1]
    return pl.pallas_call(
        _head_kernel,
        grid=(h.shape[0] // BLK,),
        in_specs=[
            pl.BlockSpec((BLK, h.shape[1]), lambda i: (i, 0)),
            pl.BlockSpec((h.shape[1], m1), lambda i: (0, 0)),
            pl.BlockSpec((1, m1), lambda i: (0, 0)),
            pl.BlockSpec((m1, m2), lambda i: (0, 0)),
            pl.BlockSpec((1, m2), lambda i: (0, 0)),
        ],
        out_specs=pl.BlockSpec((1, m2), lambda i: (0, 0)),
        out_shape=jax.ShapeDtypeStruct((1, m2), jnp.float32),
    )(h, Wl, bl.reshape(1, -1), Wl2, bl2.reshape(1, -1))


def _edge_partition_body(src_hbm, dst_hbm, lsrc_hbm, ldst_hbm, counts_hbm,
                         schunk, dchunk, lsv, ldv, cbuf):
    wid = _wid()
    base = wid * CAPW
    lo = wid * RPW
    hi = lo + RPW
    zero16 = jnp.zeros((16,), jnp.int32)

    def init_body(i, c):
        lsv[pl.ds(i * 16, 16)] = zero16
        ldv[pl.ds(i * 16, 16)] = zero16
        return c

    lax.fori_loop(0, (FLUSH + 32) // 16, init_body, 0)

    def flush(nflush):
        pltpu.sync_copy(lsv.at[pl.ds(0, FLUSH)],
                        lsrc_hbm.at[pl.ds(base + nflush * FLUSH, FLUSH)])
        pltpu.sync_copy(ldv.at[pl.ds(0, FLUSH)],
                        ldst_hbm.at[pl.ds(base + nflush * FLUSH, FLUSH)])

    def chunk_body(ci, carry):
        pltpu.sync_copy(src_hbm.at[pl.ds(ci * CH, CH)], schunk)
        pltpu.sync_copy(dst_hbm.at[pl.ds(ci * CH, CH)], dchunk)

        def grp(g, c2):
            cnt_buf, nflush = c2
            d = dchunk[pl.ds(g * 16, 16)]
            s = schunk[pl.ds(g * 16, 16)]
            m = (d >= lo) & (d < hi)
            pm = plsc.cumsum(m.astype(jnp.int32))
            pos = cnt_buf + pm - 1
            plsc.store_scatter(lsv, [pos], s, mask=m)
            plsc.store_scatter(ldv, [pos], d - lo, mask=m)
            cnt_buf = cnt_buf + pm[15]
            full = cnt_buf >= FLUSH

            @pl.when(full)
            def _():
                flush(nflush)
                t1 = lsv[pl.ds(FLUSH, 16)]
                lsv[pl.ds(0, 16)] = t1
                t2 = ldv[pl.ds(FLUSH, 16)]
                ldv[pl.ds(0, 16)] = t2

            cnt_buf = jnp.where(full, cnt_buf - FLUSH, cnt_buf)
            nflush = jnp.where(full, nflush + 1, nflush)
            return (cnt_buf, nflush)

        return lax.fori_loop(0, CH // 16, grp, carry)

    cnt_buf, nflush = lax.fori_loop(0, E // CH, chunk_body,
                                    (jnp.int32(0), jnp.int32(0)))

    @pl.when(cnt_buf > 0)
    def _():
        flush(nflush)

    total = nflush * FLUSH + cnt_buf
    cbuf[...] = jnp.full((16,), 1, jnp.int32) * total
    pltpu.sync_copy(cbuf, counts_hbm.at[pl.ds(wid * 16, 16)])


_edge_partition = functools.partial(
    pl.kernel,
    out_type=[
        jax.ShapeDtypeStruct((NW * CAPW,), jnp.int32),
        jax.ShapeDtypeStruct((NW * CAPW,), jnp.int32),
        jax.ShapeDtypeStruct((NW * 16,), jnp.int32),
    ],
    mesh=_mesh,
    compiler_params=_sc_params,
    scratch_types=[
        pltpu.VMEM((CH,), jnp.int32),
        pltpu.VMEM((CH,), jnp.int32),
        pltpu.VMEM((FLUSH + 32,), jnp.int32),
        pltpu.VMEM((FLUSH + 32,), jnp.int32),
        pltpu.VMEM((16,), jnp.int32),
    ],
)(_edge_partition_body)


def _segmin_body(b_hbm, lsrc_hbm, ldst_hbm, counts_hbm, c_hbm,
                 idxb, dlocb, rows, cbuf, acc, sem):
    wid = _wid()
    base = wid * CAPW
    infv = jnp.full((16,), jnp.inf, jnp.float32)

    def init_body(r, c):
        acc[pl.ds(r * 16, 16)] = infv
        return c

    lax.fori_loop(0, RPW * 8, init_body, 0)

    pltpu.sync_copy(counts_hbm.at[pl.ds(wid * 16, 16)], cbuf)
    cnt = cbuf[...][0]
    nblk = lax.div(cnt + (GB - 1), GB)
    iota16 = lax.iota(jnp.int32, 16)
    off8 = iota16 * 8

    def blk_body(b, c):
        pltpu.sync_copy(lsrc_hbm.at[pl.ds(base + b * GB, GB)], idxb)
        pltpu.sync_copy(ldst_hbm.at[pl.ds(base + b * GB, GB)], dlocb)
        pltpu.async_copy(b_hbm.at[idxb], rows, sem).wait()

        def sg_body(sg, c2):
            d16 = dlocb[pl.ds(sg * 16, 16)]
            a16 = d16 * HID
            r16 = sg * 16 + iota16
            valid = (b * GB + sg * 16 + iota16) < cnt
            for i in range(128):
                col = (off8 + i) & 127
                bv = plsc.load_gather(rows, [r16, col])
                av = plsc.load_gather(acc, [a16 + col], mask=valid)
                plsc.store_scatter(acc, [a16 + col], jnp.minimum(av, bv),
                                   mask=valid)
            return c2

        lax.fori_loop(0, GB // 16, sg_body, 0)
        return c

    lax.fori_loop(0, nblk, blk_body, 0)
    pltpu.sync_copy(acc, c_hbm.at[pl.ds(wid * RPW * HID, RPW * HID)])


_segmin = functools.partial(
    pl.kernel,
    out_type=jax.ShapeDtypeStruct((NPAD * HID,), jnp.float32),
    mesh=_mesh,
    compiler_params=_sc_params,
    scratch_types=[
        pltpu.VMEM((GB,), jnp.int32),
        pltpu.VMEM((GB,), jnp.int32),
        pltpu.VMEM((GB, HID), jnp.float32),
        pltpu.VMEM((16,), jnp.int32),
        pltpu.VMEM((RPW * HID,), jnp.float32),
        pltpu.SemaphoreType.DMA,
    ],
)(_segmin_body)


def kernel(x, edge_index, Wt1, bt1, Wp1, bp1, Wt2, bt2, Wp2, bp2, Wt3, bt3,
           Wp3, bp3, Wl, bl, Wl2, bl2, Wo, bo):
    src = edge_index[0]
    dst = edge_index[1]
    lsrc, ldst, counts = _edge_partition(src, dst)
    h = x
    for Wt, bt, Wp, bp in ((Wt1, bt1, Wp1, bp1), (Wt2, bt2, Wp2, bp2),
                           (Wt3, bt3, Wp3, bp3)):
        W = jnp.concatenate([Wt + Wp, Wt], axis=1)
        b = jnp.concatenate([bt + bp, jnp.zeros_like(bt)])
        AB = _dense(h, W, b)
        A = AB[:, :HID]
        B = AB[:, HID:]
        C = _segmin(B, lsrc, ldst, counts).reshape(NPAD, HID)[:N]
        h = A - C
        h = jnp.where(jnp.isfinite(h), h, 0.0)
    s = _head(h, Wl, bl, Wl2, bl2)
    out = (s / N) @ Wo + bo
    return out.reshape(-1)


# speedup vs baseline: 6.5677x; 2.8397x over previous
import functools

import jax
import jax.numpy as jnp
from jax import lax
from jax.experimental import pallas as pl
from jax.experimental.pallas import tpu as pltpu
from jax.experimental.pallas import tpu_sc as plsc

N = 10000
E = 320000
HID = 128
BLK = 1000

NW = 32
RPW = 313
NPAD = NW * RPW
CH = 2560
FLUSH = 8192
VCAP = FLUSH + 80
CAPW = 40 * FLUSH
GB = 128

_mesh = plsc.VectorSubcoreMesh(core_axis_name="c", subcore_axis_name="s")
_sc_params = pltpu.CompilerParams(needs_layout_passes=False)


def _wid():
    return lax.axis_index("s") * 2 + lax.axis_index("c")


def _mm_kernel(h_ref, w_ref, b_ref, o_ref):
    o_ref[...] = (
        jnp.dot(h_ref[...], w_ref[...], preferred_element_type=jnp.float32)
        + b_ref[...]
    )


def _dense(h, W, b):
    n, k = h.shape
    m = W.shape[1]
    return pl.pallas_call(
        _mm_kernel,
        grid=(n // BLK,),
        in_specs=[
            pl.BlockSpec((BLK, k), lambda i: (i, 0)),
            pl.BlockSpec((k, m), lambda i: (0, 0)),
            pl.BlockSpec((1, m), lambda i: (0, 0)),
        ],
        out_specs=pl.BlockSpec((BLK, m), lambda i: (i, 0)),
        out_shape=jax.ShapeDtypeStruct((n, m), jnp.float32),
    )(h, W, b.reshape(1, -1))


def _head_kernel(h_ref, wl_ref, bl_ref, wl2_ref, bl2_ref, o_ref):
    i = pl.program_id(0)
    t = jnp.dot(h_ref[...], wl_ref[...], preferred_element_type=jnp.float32)
    t = jnp.maximum(t + bl_ref[...], 0.0)
    t = jnp.dot(t, wl2_ref[...], preferred_element_type=jnp.float32)
    t = jnp.maximum(t + bl2_ref[...], 0.0)
    s = jnp.sum(t, axis=0, keepdims=True)

    @pl.when(i == 0)
    def _():
        o_ref[...] = jnp.zeros_like(o_ref)

    o_ref[...] += s


def _head(h, Wl, bl, Wl2, bl2):
    m1 = Wl.shape[1]
    m2 = Wl2.shape[1]
    return pl.pallas_call(
        _head_kernel,
        grid=(h.shape[0] // BLK,),
        in_specs=[
            pl.BlockSpec((BLK, h.shape[1]), lambda i: (i, 0)),
            pl.BlockSpec((h.shape[1], m1), lambda i: (0, 0)),
            pl.BlockSpec((1, m1), lambda i: (0, 0)),
            pl.BlockSpec((m1, m2), lambda i: (0, 0)),
            pl.BlockSpec((1, m2), lambda i: (0, 0)),
        ],
        out_specs=pl.BlockSpec((1, m2), lambda i: (0, 0)),
        out_shape=jax.ShapeDtypeStruct((1, m2), jnp.float32),
    )(h, Wl, bl.reshape(1, -1), Wl2, bl2.reshape(1, -1))


def _edge_partition_body(src_hbm, dst_hbm, lsrc_hbm, ldst_hbm, counts_hbm,
                         schunk, dchunk, lsv, ldv, cbuf, ssem, dsem):
    wid = _wid()
    base = wid * CAPW
    lo = wid * RPW
    hi = lo + RPW
    zero16 = jnp.zeros((16,), jnp.int32)
    dummy16 = jnp.full((16,), RPW, jnp.int32)

    def init_body(i, c):
        lsv[pl.ds(i * 16, 16)] = zero16
        ldv[pl.ds(i * 16, 16)] = dummy16
        return c

    lax.fori_loop(0, VCAP // 16, init_body, 0)

    def flush(nflush):
        pltpu.sync_copy(lsv.at[pl.ds(0, FLUSH)],
                        lsrc_hbm.at[pl.ds(base + nflush * FLUSH, FLUSH)])
        pltpu.sync_copy(ldv.at[pl.ds(0, FLUSH)],
                        ldst_hbm.at[pl.ds(base + nflush * FLUSH, FLUSH)])

    def start_chunk(ci, buf):
        pltpu.async_copy(src_hbm.at[pl.ds(ci * CH, CH)],
                         schunk.at[buf], ssem.at[buf])
        pltpu.async_copy(dst_hbm.at[pl.ds(ci * CH, CH)],
                         dchunk.at[buf], dsem.at[buf])

    def wait_chunk(buf):
        pltpu.make_async_copy(src_hbm.at[pl.ds(0, CH)],
                              schunk.at[buf], ssem.at[buf]).wait()
        pltpu.make_async_copy(dst_hbm.at[pl.ds(0, CH)],
                              dchunk.at[buf], dsem.at[buf]).wait()

    start_chunk(0, 0)

    def chunk_body(ci, carry):
        buf = lax.rem(ci, 2)

        @pl.when(ci + 1 < E // CH)
        def _():
            start_chunk(ci + 1, 1 - buf)

        wait_chunk(buf)

        def blk4(g4, c2):
            cnt_buf, nflush = c2
            g = g4 * 4
            pms = []
            ss = []
            ds = []
            ms = []
            for k in range(4):
                d = dchunk[buf, pl.ds((g + k) * 16, 16)]
                s = schunk[buf, pl.ds((g + k) * 16, 16)]
                m = (d >= lo) & (d < hi)
                pms.append(plsc.cumsum(m.astype(jnp.int32)))
                ss.append(s)
                ds.append(d)
                ms.append(m)
            c = cnt_buf
            for k in range(4):
                pos = c + pms[k] - 1
                plsc.store_scatter(lsv, [pos], ss[k], mask=ms[k])
                plsc.store_scatter(ldv, [pos], ds[k] - lo, mask=ms[k])
                c = c + pms[k][15]
            cnt_buf = c
            full = cnt_buf >= FLUSH

            @pl.when(full)
            def _():
                flush(nflush)
                for k in range(4):
                    t1 = lsv[pl.ds(FLUSH + k * 16, 16)]
                    lsv[pl.ds(k * 16, 16)] = t1
                    t2 = ldv[pl.ds(FLUSH + k * 16, 16)]
                    ldv[pl.ds(k * 16, 16)] = t2

            cnt_buf = jnp.where(full, cnt_buf - FLUSH, cnt_buf)
            nflush = jnp.where(full, nflush + 1, nflush)
            return (cnt_buf, nflush)

        return lax.fori_loop(0, CH // 64, blk4, carry)

    cnt_buf, nflush = lax.fori_loop(0, E // CH, chunk_body,
                                    (jnp.int32(0), jnp.int32(0)))

    @pl.when(cnt_buf > 0)
    def _():
        flush(nflush)

    total = nflush * FLUSH + cnt_buf
    cbuf[...] = jnp.full((16,), 1, jnp.int32) * total
    pltpu.sync_copy(cbuf, counts_hbm.at[pl.ds(wid * 16, 16)])


_edge_partition = functools.partial(
    pl.kernel,
    out_type=[
        jax.ShapeDtypeStruct((NW * CAPW,), jnp.int32),
        jax.ShapeDtypeStruct((NW * CAPW,), jnp.int32),
        jax.ShapeDtypeStruct((NW * 16,), jnp.int32),
    ],
    mesh=_mesh,
    compiler_params=_sc_params,
    scratch_types=[
        pltpu.VMEM((2, CH), jnp.int32),
        pltpu.VMEM((2, CH), jnp.int32),
        pltpu.VMEM((VCAP,), jnp.int32),
        pltpu.VMEM((VCAP,), jnp.int32),
        pltpu.VMEM((16,), jnp.int32),
        pltpu.SemaphoreType.DMA((2,)),
        pltpu.SemaphoreType.DMA((2,)),
    ],
)(_edge_partition_body)


def _segmin_body(b_hbm, lsrc_hbm, ldst_hbm, counts_hbm, c_hbm,
                 i0, i1, i2, d0, d1, d2, r0, r1, r2, cbuf, acc,
                 is0, is1, is2, ds0, ds1, ds2, rs0, rs1, rs2):
    wid = _wid()
    base = wid * CAPW
    infv = jnp.full((16,), jnp.inf, jnp.float32)
    idxb = (i0, i1, i2)
    dlocb = (d0, d1, d2)
    rows = (r0, r1, r2)
    isem = (is0, is1, is2)
    dsem = (ds0, ds1, ds2)
    rsem = (rs0, rs1, rs2)

    def init_body(r, c):
        acc[pl.ds(r * 16, 16)] = infv
        return c

    lax.fori_loop(0, (RPW + 1) * 8, init_body, 0)

    pltpu.sync_copy(counts_hbm.at[pl.ds(wid * 16, 16)], cbuf)
    cnt = cbuf[...][0]
    nblk = lax.div(cnt + (GB - 1), GB)

    def start_idx(b, s):
        pltpu.async_copy(lsrc_hbm.at[pl.ds(base + b * GB, GB)],
                         idxb[s], isem[s])
        pltpu.async_copy(ldst_hbm.at[pl.ds(base + b * GB, GB)],
                         dlocb[s], dsem[s])

    def wait_idx(s):
        pltpu.make_async_copy(lsrc_hbm.at[pl.ds(0, GB)],
                              idxb[s], isem[s]).wait()
        pltpu.make_async_copy(ldst_hbm.at[pl.ds(0, GB)],
                              dlocb[s], dsem[s]).wait()

    def start_rows(s):
        pltpu.async_copy(b_hbm.at[idxb[s]], rows[s], rsem[s])

    def wait_rows(s):
        pltpu.make_async_copy(b_hbm.at[pl.ds(0, GB)],
                              rows[s], rsem[s]).wait()

    def compute(s):
        rr = rows[s]
        dd = dlocb[s]

        def sg_body(sg, c2):
            a16 = dd[pl.ds(sg * 16, 16)] * HID
            for j in range(16):
                a = a16[j]
                e = sg * 16 + j
                for v in range(8):
                    av = acc[pl.ds(a + v * 16, 16)]
                    bv = rr[e, pl.ds(v * 16, 16)]
                    acc[pl.ds(a + v * 16, 16)] = jnp.minimum(av, bv)
            return c2

        lax.fori_loop(0, GB // 16, sg_body, 0)

    def do_block(b, s):
        s1 = (s + 1) % 3
        s2 = (s + 2) % 3

        @pl.when(b + 1 < nblk)
        def _():
            wait_idx(s1)
            start_rows(s1)

        wait_rows(s)

        @pl.when(b + 2 < nblk)
        def _():
            start_idx(b + 2, s2)

        compute(s)

    @pl.when(nblk > 0)
    def _():
        start_idx(0, 0)

    @pl.when(nblk > 1)
    def _():
        start_idx(1, 1)

    @pl.when(nblk > 0)
    def _():
        wait_idx(0)
        start_rows(0)

    def macro_body(g, c):
        b = g * 3
        for k in range(3):
            @pl.when(b + k < nblk)
            def _(k=k):
                do_block(b + k, k)
        return c

    lax.fori_loop(0, lax.div(nblk + 2, 3), macro_body, 0)
    pltpu.sync_copy(acc.at[pl.ds(0, RPW * HID)],
                    c_hbm.at[pl.ds(wid * RPW * HID, RPW * HID)])


_segmin = functools.partial(
    pl.kernel,
    out_type=jax.ShapeDtypeStruct((NPAD * HID,), jnp.float32),
    mesh=_mesh,
    compiler_params=_sc_params,
    scratch_types=(
        [pltpu.VMEM((GB,), jnp.int32)] * 6
        + [pltpu.VMEM((GB, HID), jnp.float32)] * 3
        + [pltpu.VMEM((16,), jnp.int32),
           pltpu.VMEM(((RPW + 1) * HID,), jnp.float32)]
        + [pltpu.SemaphoreType.DMA] * 9
    ),
)(_segmin_body)


def kernel(x, edge_index, Wt1, bt1, Wp1, bp1, Wt2, bt2, Wp2, bp2, Wt3, bt3,
           Wp3, bp3, Wl, bl, Wl2, bl2, Wo, bo):
    src = edge_index[0]
    dst = edge_index[1]
    lsrc, ldst, counts = _edge_partition(src, dst)
    h = x
    for Wt, bt, Wp, bp in ((Wt1, bt1, Wp1, bp1), (Wt2, bt2, Wp2, bp2),
                           (Wt3, bt3, Wp3, bp3)):
        W = jnp.concatenate([Wt + Wp, Wt], axis=1)
        b = jnp.concatenate([bt + bp, jnp.zeros_like(bt)])
        AB = _dense(h, W, b)
        A = AB[:, :HID]
        B = AB[:, HID:]
        C = _segmin(B, lsrc, ldst, counts).reshape(NPAD, HID)[:N]
        h = A - C
        h = jnp.where(jnp.isfinite(h), h, 0.0)
    s = _head(h, Wl, bl, Wl2, bl2)
    out = (s / N) @ Wo + bo
    return out.reshape(-1)


# speedup vs baseline: 11.7202x; 1.7845x over previous
import functools

import jax
import jax.numpy as jnp
from jax import lax
from jax.experimental import pallas as pl
from jax.experimental.pallas import tpu as pltpu
from jax.experimental.pallas import tpu_sc as plsc

N = 10000
E = 320000
HID = 128
BLK = 1000

NW = 32
RPW = 313
NPAD = NW * RPW
CH = 2560
FLUSH = 8192
VCAP = FLUSH + 80
CAPW = 40 * FLUSH
GB = 128

_mesh = plsc.VectorSubcoreMesh(core_axis_name="c", subcore_axis_name="s")
_sc_params = pltpu.CompilerParams(needs_layout_passes=False)


def _wid():
    return lax.axis_index("s") * 2 + lax.axis_index("c")


def _mm_kernel(h_ref, w_ref, b_ref, o_ref):
    o_ref[...] = (
        jnp.dot(h_ref[...], w_ref[...], preferred_element_type=jnp.float32)
        + b_ref[...]
    )


def _dense(h, W, b):
    n, k = h.shape
    m = W.shape[1]
    return pl.pallas_call(
        _mm_kernel,
        grid=(n // BLK,),
        in_specs=[
            pl.BlockSpec((BLK, k), lambda i: (i, 0)),
            pl.BlockSpec((k, m), lambda i: (0, 0)),
            pl.BlockSpec((1, m), lambda i: (0, 0)),
        ],
        out_specs=pl.BlockSpec((BLK, m), lambda i: (i, 0)),
        out_shape=jax.ShapeDtypeStruct((n, m), jnp.float32),
    )(h, W, b.reshape(1, -1))


def _head_kernel(h_ref, wl_ref, bl_ref, wl2_ref, bl2_ref, o_ref):
    i = pl.program_id(0)
    t = jnp.dot(h_ref[...], wl_ref[...], preferred_element_type=jnp.float32)
    t = jnp.maximum(t + bl_ref[...], 0.0)
    t = jnp.dot(t, wl2_ref[...], preferred_element_type=jnp.float32)
    t = jnp.maximum(t + bl2_ref[...], 0.0)
    s = jnp.sum(t, axis=0, keepdims=True)

    @pl.when(i == 0)
    def _():
        o_ref[...] = jnp.zeros_like(o_ref)

    o_ref[...] += s


def _head(h, Wl, bl, Wl2, bl2):
    m1 = Wl.shape[1]
    m2 = Wl2.shape[1]
    return pl.pallas_call(
        _head_kernel,
        grid=(h.shape[0] // BLK,),
        in_specs=[
            pl.BlockSpec((BLK, h.shape[1]), lambda i: (i, 0)),
            pl.BlockSpec((h.shape[1], m1), lambda i: (0, 0)),
            pl.BlockSpec((1, m1), lambda i: (0, 0)),
            pl.BlockSpec((m1, m2), lambda i: (0, 0)),
            pl.BlockSpec((1, m2), lambda i: (0, 0)),
        ],
        out_specs=pl.BlockSpec((1, m2), lambda i: (0, 0)),
        out_shape=jax.ShapeDtypeStruct((1, m2), jnp.float32),
    )(h, Wl, bl.reshape(1, -1), Wl2, bl2.reshape(1, -1))


def _edge_partition_body(src_hbm, dst_hbm, lsrc_hbm, ldst_hbm, counts_hbm,
                         schunk, dchunk, lsv, ldv, cbuf, ssem, dsem):
    wid = _wid()
    base = wid * CAPW
    lo = wid * RPW
    hi = lo + RPW
    zero16 = jnp.zeros((16,), jnp.int32)
    dummy16 = jnp.full((16,), RPW, jnp.int32)

    def init_body(i, c):
        lsv[pl.ds(i * 16, 16)] = zero16
        ldv[pl.ds(i * 16, 16)] = dummy16
        return c

    lax.fori_loop(0, VCAP // 16, init_body, 0)

    def flush(nflush):
        pltpu.sync_copy(lsv.at[pl.ds(0, FLUSH)],
                        lsrc_hbm.at[pl.ds(base + nflush * FLUSH, FLUSH)])
        pltpu.sync_copy(ldv.at[pl.ds(0, FLUSH)],
                        ldst_hbm.at[pl.ds(base + nflush * FLUSH, FLUSH)])

    def start_chunk(ci, buf):
        pltpu.async_copy(src_hbm.at[pl.ds(ci * CH, CH)],
                         schunk.at[buf], ssem.at[buf])
        pltpu.async_copy(dst_hbm.at[pl.ds(ci * CH, CH)],
                         dchunk.at[buf], dsem.at[buf])

    def wait_chunk(buf):
        pltpu.make_async_copy(src_hbm.at[pl.ds(0, CH)],
                              schunk.at[buf], ssem.at[buf]).wait()
        pltpu.make_async_copy(dst_hbm.at[pl.ds(0, CH)],
                              dchunk.at[buf], dsem.at[buf]).wait()

    start_chunk(0, 0)

    def chunk_body(ci, carry):
        buf = lax.rem(ci, 2)

        @pl.when(ci + 1 < E // CH)
        def _():
            start_chunk(ci + 1, 1 - buf)

        wait_chunk(buf)

        def blk4(g4, c2):
            cnt_buf, nflush = c2
            g = g4 * 4
            pms = []
            ss = []
            ds = []
            ms = []
            for k in range(4):
                d = dchunk[buf, pl.ds((g + k) * 16, 16)]
                s = schunk[buf, pl.ds((g + k) * 16, 16)]
                m = (d >= lo) & (d < hi)
                pms.append(plsc.cumsum(m.astype(jnp.int32)))
                ss.append(s)
                ds.append(d)
                ms.append(m)
            c = cnt_buf
            for k in range(4):
                pos = c + pms[k] - 1
                plsc.store_scatter(lsv, [pos], ss[k], mask=ms[k])
                plsc.store_scatter(ldv, [pos], ds[k] - lo, mask=ms[k])
                c = c + pms[k][15]
            cnt_buf = c
            full = cnt_buf >= FLUSH

            @pl.when(full)
            def _():
                flush(nflush)
                for k in range(4):
                    t1 = lsv[pl.ds(FLUSH + k * 16, 16)]
                    lsv[pl.ds(k * 16, 16)] = t1
                    t2 = ldv[pl.ds(FLUSH + k * 16, 16)]
                    ldv[pl.ds(k * 16, 16)] = t2

            cnt_buf = jnp.where(full, cnt_buf - FLUSH, cnt_buf)
            nflush = jnp.where(full, nflush + 1, nflush)
            return (cnt_buf, nflush)

        return lax.fori_loop(0, CH // 64, blk4, carry)

    cnt_buf, nflush = lax.fori_loop(0, E // CH, chunk_body,
                                    (jnp.int32(0), jnp.int32(0)))

    @pl.when(cnt_buf > 0)
    def _():
        flush(nflush)

    total = nflush * FLUSH + cnt_buf
    cbuf[...] = jnp.full((16,), 1, jnp.int32) * total
    pltpu.sync_copy(cbuf, counts_hbm.at[pl.ds(wid * 16, 16)])


_edge_partition = functools.partial(
    pl.kernel,
    out_type=[
        jax.ShapeDtypeStruct((NW * CAPW,), jnp.int32),
        jax.ShapeDtypeStruct((NW * CAPW,), jnp.int32),
        jax.ShapeDtypeStruct((NW * 16,), jnp.int32),
    ],
    mesh=_mesh,
    compiler_params=_sc_params,
    scratch_types=[
        pltpu.VMEM((2, CH), jnp.int32),
        pltpu.VMEM((2, CH), jnp.int32),
        pltpu.VMEM((VCAP,), jnp.int32),
        pltpu.VMEM((VCAP,), jnp.int32),
        pltpu.VMEM((16,), jnp.int32),
        pltpu.SemaphoreType.DMA((2,)),
        pltpu.SemaphoreType.DMA((2,)),
    ],
)(_edge_partition_body)


def _segmin_body(b_hbm, lsrc_hbm, ldst_hbm, counts_hbm, c_hbm,
                 i0, i1, i2, d0, d1, d2, r0, r1, r2, cbuf, acc,
                 is0, is1, is2, ds0, ds1, ds2, rs0, rs1, rs2):
    wid = _wid()
    base = wid * CAPW
    infv = jnp.full((16,), jnp.inf, jnp.float32)
    idxb = (i0, i1, i2)
    dlocb = (d0, d1, d2)
    rows = (r0, r1, r2)
    isem = (is0, is1, is2)
    dsem = (ds0, ds1, ds2)
    rsem = (rs0, rs1, rs2)

    def init_body(r, c):
        acc[pl.ds(r * 16, 16)] = infv
        return c

    lax.fori_loop(0, (RPW + 1) * 8, init_body, 0)

    pltpu.sync_copy(counts_hbm.at[pl.ds(wid * 16, 16)], cbuf)
    cnt = cbuf[...][0]
    nblk = lax.div(cnt + (GB - 1), GB)

    def start_idx(b, s):
        pltpu.async_copy(lsrc_hbm.at[pl.ds(base + b * GB, GB)],
                         idxb[s], isem[s])
        pltpu.async_copy(ldst_hbm.at[pl.ds(base + b * GB, GB)],
                         dlocb[s], dsem[s])

    def wait_idx(s):
        pltpu.make_async_copy(lsrc_hbm.at[pl.ds(0, GB)],
                              idxb[s], isem[s]).wait()
        pltpu.make_async_copy(ldst_hbm.at[pl.ds(0, GB)],
                              dlocb[s], dsem[s]).wait()

    def start_rows(s):
        pltpu.async_copy(b_hbm.at[idxb[s]], rows[s], rsem[s])

    def wait_rows(s):
        pltpu.make_async_copy(b_hbm.at[pl.ds(0, GB)],
                              rows[s], rsem[s]).wait()

    def compute(s):
        rr = rows[s]
        dd = dlocb[s]

        def sg_body(sg, c2):
            a16 = dd[pl.ds(sg * 16, 16)] * HID
            for j in range(16):
                a = a16[j]
                e = sg * 16 + j
                bvs = [rr[e, pl.ds(v * 16, 16)] for v in range(8)]
                avs = [acc[pl.ds(a + v * 16, 16)] for v in range(8)]
                mns = [jnp.minimum(avs[v], bvs[v]) for v in range(8)]
                for v in range(8):
                    acc[pl.ds(a + v * 16, 16)] = mns[v]
            return c2

        lax.fori_loop(0, GB // 16, sg_body, 0)

    def do_block(b, s):
        s1 = (s + 1) % 3
        s2 = (s + 2) % 3

        @pl.when(b + 1 < nblk)
        def _():
            wait_idx(s1)
            start_rows(s1)

        wait_rows(s)

        @pl.when(b + 2 < nblk)
        def _():
            start_idx(b + 2, s2)

        compute(s)

    @pl.when(nblk > 0)
    def _():
        start_idx(0, 0)

    @pl.when(nblk > 1)
    def _():
        start_idx(1, 1)

    @pl.when(nblk > 0)
    def _():
        wait_idx(0)
        start_rows(0)

    def macro_body(g, c):
        b = g * 3
        for k in range(3):
            @pl.when(b + k < nblk)
            def _(k=k):
                do_block(b + k, k)
        return c

    lax.fori_loop(0, lax.div(nblk + 2, 3), macro_body, 0)
    pltpu.sync_copy(acc.at[pl.ds(0, RPW * HID)],
                    c_hbm.at[pl.ds(wid * RPW * HID, RPW * HID)])


_segmin = functools.partial(
    pl.kernel,
    out_type=jax.ShapeDtypeStruct((NPAD * HID,), jnp.float32),
    mesh=_mesh,
    compiler_params=_sc_params,
    scratch_types=(
        [pltpu.VMEM((GB,), jnp.int32)] * 6
        + [pltpu.VMEM((GB, HID), jnp.float32)] * 3
        + [pltpu.VMEM((16,), jnp.int32),
           pltpu.VMEM(((RPW + 1) * HID,), jnp.float32)]
        + [pltpu.SemaphoreType.DMA] * 9
    ),
)(_segmin_body)


def kernel(x, edge_index, Wt1, bt1, Wp1, bp1, Wt2, bt2, Wp2, bp2, Wt3, bt3,
           Wp3, bp3, Wl, bl, Wl2, bl2, Wo, bo):
    src = edge_index[0]
    dst = edge_index[1]
    lsrc, ldst, counts = _edge_partition(src, dst)
    h = x
    for Wt, bt, Wp, bp in ((Wt1, bt1, Wp1, bp1), (Wt2, bt2, Wp2, bp2),
                           (Wt3, bt3, Wp3, bp3)):
        W = jnp.concatenate([Wt + Wp, Wt], axis=1)
        b = jnp.concatenate([bt + bp, jnp.zeros_like(bt)])
        AB = _dense(h, W, b)
        A = AB[:, :HID]
        B = AB[:, HID:]
        C = _segmin(B, lsrc, ldst, counts).reshape(NPAD, HID)[:N]
        h = A - C
        h = jnp.where(jnp.isfinite(h), h, 0.0)
    s = _head(h, Wl, bl, Wl2, bl2)
    out = (s / N) @ Wo + bo
    return out.reshape(-1)


# speedup vs baseline: 11.8913x; 1.0146x over previous
import functools

import jax
import jax.numpy as jnp
from jax import lax
from jax.experimental import pallas as pl
from jax.experimental.pallas import tpu as pltpu
from jax.experimental.pallas import tpu_sc as plsc

N = 10000
E = 320000
HID = 128
BLK = 1000

NW = 32
RPW = 313
NPAD = NW * RPW
CH = 2560
FLUSH = 8192
VCAP = FLUSH + 80
CAPW = 40 * FLUSH
GB = 128

_mesh = plsc.VectorSubcoreMesh(core_axis_name="c", subcore_axis_name="s")
_sc_params = pltpu.CompilerParams(needs_layout_passes=False)


def _wid():
    return lax.axis_index("s") * 2 + lax.axis_index("c")


def _mm_kernel(h_ref, w_ref, b_ref, a_ref, bb_ref):
    ab = (
        jnp.dot(h_ref[...], w_ref[...], preferred_element_type=jnp.float32)
        + b_ref[...]
    )
    a_ref[...] = ab[:, :HID]
    bb_ref[...] = ab[:, HID:]


def _dense(h, W, b):
    n, k = h.shape
    return pl.pallas_call(
        _mm_kernel,
        grid=(n // BLK,),
        in_specs=[
            pl.BlockSpec((BLK, k), lambda i: (i, 0)),
            pl.BlockSpec((k, 2 * HID), lambda i: (0, 0)),
            pl.BlockSpec((1, 2 * HID), lambda i: (0, 0)),
        ],
        out_specs=[
            pl.BlockSpec((BLK, HID), lambda i: (i, 0)),
            pl.BlockSpec((BLK, HID), lambda i: (i, 0)),
        ],
        out_shape=[
            jax.ShapeDtypeStruct((n, HID), jnp.float32),
            jax.ShapeDtypeStruct((n, HID), jnp.float32),
        ],
    )(h, W, b.reshape(1, -1))


def _mm_fused_kernel(a_ref, c_ref, w_ref, b_ref, ao_ref, bo_ref):
    h = a_ref[...] - c_ref[...]
    h = jnp.where(jnp.isfinite(h), h, 0.0)
    ab = (
        jnp.dot(h, w_ref[...], preferred_element_type=jnp.float32)
        + b_ref[...]
    )
    ao_ref[...] = ab[:, :HID]
    bo_ref[...] = ab[:, HID:]


def _dense_fused(A, C, W, b):
    n = A.shape[0]
    return pl.pallas_call(
        _mm_fused_kernel,
        grid=(n // BLK,),
        in_specs=[
            pl.BlockSpec((BLK, HID), lambda i: (i, 0)),
            pl.BlockSpec((BLK, HID), lambda i: (i, 0)),
            pl.BlockSpec((HID, 2 * HID), lambda i: (0, 0)),
            pl.BlockSpec((1, 2 * HID), lambda i: (0, 0)),
        ],
        out_specs=[
            pl.BlockSpec((BLK, HID), lambda i: (i, 0)),
            pl.BlockSpec((BLK, HID), lambda i: (i, 0)),
        ],
        out_shape=[
            jax.ShapeDtypeStruct((n, HID), jnp.float32),
            jax.ShapeDtypeStruct((n, HID), jnp.float32),
        ],
    )(A, C, W, b.reshape(1, -1))


def _head_kernel(a_ref, c_ref, wl_ref, bl_ref, wl2_ref, bl2_ref, o_ref):
    i = pl.program_id(0)
    h = a_ref[...] - c_ref[...]
    h = jnp.where(jnp.isfinite(h), h, 0.0)
    t = jnp.dot(h, wl_ref[...], preferred_element_type=jnp.float32)
    t = jnp.maximum(t + bl_ref[...], 0.0)
    t = jnp.dot(t, wl2_ref[...], preferred_element_type=jnp.float32)
    t = jnp.maximum(t + bl2_ref[...], 0.0)
    s = jnp.sum(t, axis=0, keepdims=True)

    @pl.when(i == 0)
    def _():
        o_ref[...] = jnp.zeros_like(o_ref)

    o_ref[...] += s


def _head(A, C, Wl, bl, Wl2, bl2):
    m1 = Wl.shape[1]
    m2 = Wl2.shape[1]
    return pl.pallas_call(
        _head_kernel,
        grid=(A.shape[0] // BLK,),
        in_specs=[
            pl.BlockSpec((BLK, HID), lambda i: (i, 0)),
            pl.BlockSpec((BLK, HID), lambda i: (i, 0)),
            pl.BlockSpec((HID, m1), lambda i: (0, 0)),
            pl.BlockSpec((1, m1), lambda i: (0, 0)),
            pl.BlockSpec((m1, m2), lambda i: (0, 0)),
            pl.BlockSpec((1, m2), lambda i: (0, 0)),
        ],
        out_specs=pl.BlockSpec((1, m2), lambda i: (0, 0)),
        out_shape=jax.ShapeDtypeStruct((1, m2), jnp.float32),
    )(A, C, Wl, bl.reshape(1, -1), Wl2, bl2.reshape(1, -1))


def _edge_partition_body(src_hbm, dst_hbm, lsrc_hbm, ldst_hbm, counts_hbm,
                         schunk, dchunk, lsv, ldv, cbuf, ssem, dsem):
    wid = _wid()
    base = wid * CAPW
    lo = wid * RPW
    hi = lo + RPW
    zero16 = jnp.zeros((16,), jnp.int32)
    dummy16 = jnp.full((16,), RPW, jnp.int32)

    def init_body(i, c):
        lsv[pl.ds(i * 16, 16)] = zero16
        ldv[pl.ds(i * 16, 16)] = dummy16
        return c

    lax.fori_loop(0, VCAP // 16, init_body, 0)

    def flush(nflush):
        pltpu.sync_copy(lsv.at[pl.ds(0, FLUSH)],
                        lsrc_hbm.at[pl.ds(base + nflush * FLUSH, FLUSH)])
        pltpu.sync_copy(ldv.at[pl.ds(0, FLUSH)],
                        ldst_hbm.at[pl.ds(base + nflush * FLUSH, FLUSH)])

    def start_chunk(ci, buf):
        pltpu.async_copy(src_hbm.at[pl.ds(ci * CH, CH)],
                         schunk.at[buf], ssem.at[buf])
        pltpu.async_copy(dst_hbm.at[pl.ds(ci * CH, CH)],
                         dchunk.at[buf], dsem.at[buf])

    def wait_chunk(buf):
        pltpu.make_async_copy(src_hbm.at[pl.ds(0, CH)],
                              schunk.at[buf], ssem.at[buf]).wait()
        pltpu.make_async_copy(dst_hbm.at[pl.ds(0, CH)],
                              dchunk.at[buf], dsem.at[buf]).wait()

    start_chunk(0, 0)

    def chunk_body(ci, carry):
        buf = lax.rem(ci, 2)

        @pl.when(ci + 1 < E // CH)
        def _():
            start_chunk(ci + 1, 1 - buf)

        wait_chunk(buf)

        def blk4(g4, c2):
            cnt_buf, nflush = c2
            g = g4 * 4
            pms = []
            ss = []
            ds = []
            ms = []
            for k in range(4):
                d = dchunk[buf, pl.ds((g + k) * 16, 16)]
                s = schunk[buf, pl.ds((g + k) * 16, 16)]
                m = (d >= lo) & (d < hi)
                pms.append(plsc.cumsum(m.astype(jnp.int32)))
                ss.append(s)
                ds.append(d)
                ms.append(m)
            c = cnt_buf
            for k in range(4):
                pos = c + pms[k] - 1
                plsc.store_scatter(lsv, [pos], ss[k], mask=ms[k])
                plsc.store_scatter(ldv, [pos], ds[k] - lo, mask=ms[k])
                c = c + pms[k][15]
            cnt_buf = c
            full = cnt_buf >= FLUSH

            @pl.when(full)
            def _():
                flush(nflush)
                for k in range(4):
                    t1 = lsv[pl.ds(FLUSH + k * 16, 16)]
                    lsv[pl.ds(k * 16, 16)] = t1
                    t2 = ldv[pl.ds(FLUSH + k * 16, 16)]
                    ldv[pl.ds(k * 16, 16)] = t2

            cnt_buf = jnp.where(full, cnt_buf - FLUSH, cnt_buf)
            nflush = jnp.where(full, nflush + 1, nflush)
            return (cnt_buf, nflush)

        return lax.fori_loop(0, CH // 64, blk4, carry)

    cnt_buf, nflush = lax.fori_loop(0, E // CH, chunk_body,
                                    (jnp.int32(0), jnp.int32(0)))

    @pl.when(cnt_buf > 0)
    def _():
        flush(nflush)

    total = nflush * FLUSH + cnt_buf
    cbuf[...] = jnp.full((16,), 1, jnp.int32) * total
    pltpu.sync_copy(cbuf, counts_hbm.at[pl.ds(wid * 16, 16)])


_edge_partition = functools.partial(
    pl.kernel,
    out_type=[
        jax.ShapeDtypeStruct((NW * CAPW,), jnp.int32),
        jax.ShapeDtypeStruct((NW * CAPW,), jnp.int32),
        jax.ShapeDtypeStruct((NW * 16,), jnp.int32),
    ],
    mesh=_mesh,
    compiler_params=_sc_params,
    scratch_types=[
        pltpu.VMEM((2, CH), jnp.int32),
        pltpu.VMEM((2, CH), jnp.int32),
        pltpu.VMEM((VCAP,), jnp.int32),
        pltpu.VMEM((VCAP,), jnp.int32),
        pltpu.VMEM((16,), jnp.int32),
        pltpu.SemaphoreType.DMA((2,)),
        pltpu.SemaphoreType.DMA((2,)),
    ],
)(_edge_partition_body)


def _segmin_body(b_hbm, lsrc_hbm, ldst_hbm, counts_hbm, c_hbm,
                 i0, i1, i2, d0, d1, d2, r0, r1, r2, cbuf, acc,
                 is0, is1, is2, ds0, ds1, ds2, rs0, rs1, rs2):
    wid = _wid()
    base = wid * CAPW
    infv = jnp.full((16,), jnp.inf, jnp.float32)
    idxb = (i0, i1, i2)
    dlocb = (d0, d1, d2)
    rows = (r0, r1, r2)
    isem = (is0, is1, is2)
    dsem = (ds0, ds1, ds2)
    rsem = (rs0, rs1, rs2)

    def init_body(r, c):
        acc[pl.ds(r * 16, 16)] = infv
        return c

    lax.fori_loop(0, (RPW + 1) * 8, init_body, 0)

    pltpu.sync_copy(counts_hbm.at[pl.ds(wid * 16, 16)], cbuf)
    cnt = cbuf[...][0]
    nblk = lax.div(cnt + (GB - 1), GB)

    def start_idx(b, s):
        pltpu.async_copy(lsrc_hbm.at[pl.ds(base + b * GB, GB)],
                         idxb[s], isem[s])
        pltpu.async_copy(ldst_hbm.at[pl.ds(base + b * GB, GB)],
                         dlocb[s], dsem[s])

    def wait_idx(s):
        pltpu.make_async_copy(lsrc_hbm.at[pl.ds(0, GB)],
                              idxb[s], isem[s]).wait()
        pltpu.make_async_copy(ldst_hbm.at[pl.ds(0, GB)],
                              dlocb[s], dsem[s]).wait()

    def start_rows(s):
        pltpu.async_copy(b_hbm.at[idxb[s]], rows[s], rsem[s])

    def wait_rows(s):
        pltpu.make_async_copy(b_hbm.at[pl.ds(0, GB)],
                              rows[s], rsem[s]).wait()

    def compute(s):
        rr = rows[s]
        dd = dlocb[s]

        def sg_body(sg, c2):
            a16 = dd[pl.ds(sg * 16, 16)] * HID
            for j in range(16):
                a = a16[j]
                e = sg * 16 + j
                bvs = [rr[e, pl.ds(v * 16, 16)] for v in range(8)]
                avs = [acc[pl.ds(a + v * 16, 16)] for v in range(8)]
                mns = [jnp.minimum(avs[v], bvs[v]) for v in range(8)]
                for v in range(8):
                    acc[pl.ds(a + v * 16, 16)] = mns[v]
            return c2

        lax.fori_loop(0, GB // 16, sg_body, 0)

    def do_block(b, s):
        s1 = (s + 1) % 3
        s2 = (s + 2) % 3

        @pl.when(b + 1 < nblk)
        def _():
            wait_idx(s1)
            start_rows(s1)

        wait_rows(s)

        @pl.when(b + 2 < nblk)
        def _():
            start_idx(b + 2, s2)

        compute(s)

    @pl.when(nblk > 0)
    def _():
        start_idx(0, 0)

    @pl.when(nblk > 1)
    def _():
        start_idx(1, 1)

    @pl.when(nblk > 0)
    def _():
        wait_idx(0)
        start_rows(0)

    def macro_body(g, c):
        b = g * 3
        for k in range(3):
            @pl.when(b + k < nblk)
            def _(k=k):
                do_block(b + k, k)
        return c

    lax.fori_loop(0, lax.div(nblk + 2, 3), macro_body, 0)
    pltpu.sync_copy(acc.at[pl.ds(0, RPW * HID)],
                    c_hbm.at[pl.ds(wid * RPW * HID, RPW * HID)])


_segmin = functools.partial(
    pl.kernel,
    out_type=jax.ShapeDtypeStruct((NPAD * HID,), jnp.float32),
    mesh=_mesh,
    compiler_params=_sc_params,
    scratch_types=(
        [pltpu.VMEM((GB,), jnp.int32)] * 6
        + [pltpu.VMEM((GB, HID), jnp.float32)] * 3
        + [pltpu.VMEM((16,), jnp.int32),
           pltpu.VMEM(((RPW + 1) * HID,), jnp.float32)]
        + [pltpu.SemaphoreType.DMA] * 9
    ),
)(_segmin_body)


def kernel(x, edge_index, Wt1, bt1, Wp1, bp1, Wt2, bt2, Wp2, bp2, Wt3, bt3,
           Wp3, bp3, Wl, bl, Wl2, bl2, Wo, bo):
    src = edge_index[0]
    dst = edge_index[1]
    lsrc, ldst, counts = _edge_partition(src, dst)
    ws = ((Wt1, bt1, Wp1, bp1), (Wt2, bt2, Wp2, bp2), (Wt3, bt3, Wp3, bp3))

    def wb(Wt, bt, Wp, bp):
        return (jnp.concatenate([Wt + Wp, Wt], axis=1),
                jnp.concatenate([bt + bp, jnp.zeros_like(bt)]))

    W, b = wb(*ws[0])
    A, B = _dense(x, W, b)
    for lyr in (1, 2):
        C = _segmin(B, lsrc, ldst, counts).reshape(NPAD, HID)[:N]
        W, b = wb(*ws[lyr])
        A, B = _dense_fused(A, C, W, b)
    C = _segmin(B, lsrc, ldst, counts).reshape(NPAD, HID)[:N]
    s = _head(A, C, Wl, bl, Wl2, bl2)
    out = (s / N) @ Wo + bo
    return out.reshape(-1)


# speedup vs baseline: 13.0342x; 1.0961x over previous
import functools

import jax
import jax.numpy as jnp
from jax import lax
from jax.experimental import pallas as pl
from jax.experimental.pallas import tpu as pltpu
from jax.experimental.pallas import tpu_sc as plsc

N = 10000
E = 320000
HID = 128
BLK = 1000

NW = 32
RPW = 313
NPAD = NW * RPW
CH = 2560
FLUSH = 8192
VCAP = FLUSH + 144
CAPW = 40 * FLUSH
GB = 128

_mesh = plsc.VectorSubcoreMesh(core_axis_name="c", subcore_axis_name="s")
_sc_params = pltpu.CompilerParams(needs_layout_passes=False)


def _wid():
    return lax.axis_index("s") * 2 + lax.axis_index("c")


def _mm_kernel(h_ref, w_ref, b_ref, a_ref, bb_ref):
    ab = (
        jnp.dot(h_ref[...], w_ref[...], preferred_element_type=jnp.float32)
        + b_ref[...]
    )
    a_ref[...] = ab[:, :HID]
    bb_ref[...] = ab[:, HID:]


def _dense(h, W, b):
    n, k = h.shape
    return pl.pallas_call(
        _mm_kernel,
        grid=(n // BLK,),
        in_specs=[
            pl.BlockSpec((BLK, k), lambda i: (i, 0)),
            pl.BlockSpec((k, 2 * HID), lambda i: (0, 0)),
            pl.BlockSpec((1, 2 * HID), lambda i: (0, 0)),
        ],
        out_specs=[
            pl.BlockSpec((BLK, HID), lambda i: (i, 0)),
            pl.BlockSpec((BLK, HID), lambda i: (i, 0)),
        ],
        out_shape=[
            jax.ShapeDtypeStruct((n, HID), jnp.float32),
            jax.ShapeDtypeStruct((n, HID), jnp.float32),
        ],
    )(h, W, b.reshape(1, -1))


def _mm_fused_kernel(a_ref, c_ref, w_ref, b_ref, ao_ref, bo_ref):
    h = a_ref[...] - c_ref[...]
    h = jnp.where(jnp.isfinite(h), h, 0.0)
    ab = (
        jnp.dot(h, w_ref[...], preferred_element_type=jnp.float32)
        + b_ref[...]
    )
    ao_ref[...] = ab[:, :HID]
    bo_ref[...] = ab[:, HID:]


def _dense_fused(A, C, W, b):
    n = A.shape[0]
    return pl.pallas_call(
        _mm_fused_kernel,
        grid=(n // BLK,),
        in_specs=[
            pl.BlockSpec((BLK, HID), lambda i: (i, 0)),
            pl.BlockSpec((BLK, HID), lambda i: (i, 0)),
            pl.BlockSpec((HID, 2 * HID), lambda i: (0, 0)),
            pl.BlockSpec((1, 2 * HID), lambda i: (0, 0)),
        ],
        out_specs=[
            pl.BlockSpec((BLK, HID), lambda i: (i, 0)),
            pl.BlockSpec((BLK, HID), lambda i: (i, 0)),
        ],
        out_shape=[
            jax.ShapeDtypeStruct((n, HID), jnp.float32),
            jax.ShapeDtypeStruct((n, HID), jnp.float32),
        ],
    )(A, C, W, b.reshape(1, -1))


def _head_kernel(a_ref, c_ref, wl_ref, bl_ref, wl2_ref, bl2_ref, o_ref):
    i = pl.program_id(0)
    h = a_ref[...] - c_ref[...]
    h = jnp.where(jnp.isfinite(h), h, 0.0)
    t = jnp.dot(h, wl_ref[...], preferred_element_type=jnp.float32)
    t = jnp.maximum(t + bl_ref[...], 0.0)
    t = jnp.dot(t, wl2_ref[...], preferred_element_type=jnp.float32)
    t = jnp.maximum(t + bl2_ref[...], 0.0)
    s = jnp.sum(t, axis=0, keepdims=True)

    @pl.when(i == 0)
    def _():
        o_ref[...] = jnp.zeros_like(o_ref)

    o_ref[...] += s


def _head(A, C, Wl, bl, Wl2, bl2):
    m1 = Wl.shape[1]
    m2 = Wl2.shape[1]
    return pl.pallas_call(
        _head_kernel,
        grid=(A.shape[0] // BLK,),
        in_specs=[
            pl.BlockSpec((BLK, HID), lambda i: (i, 0)),
            pl.BlockSpec((BLK, HID), lambda i: (i, 0)),
            pl.BlockSpec((HID, m1), lambda i: (0, 0)),
            pl.BlockSpec((1, m1), lambda i: (0, 0)),
            pl.BlockSpec((m1, m2), lambda i: (0, 0)),
            pl.BlockSpec((1, m2), lambda i: (0, 0)),
        ],
        out_specs=pl.BlockSpec((1, m2), lambda i: (0, 0)),
        out_shape=jax.ShapeDtypeStruct((1, m2), jnp.float32),
    )(A, C, Wl, bl.reshape(1, -1), Wl2, bl2.reshape(1, -1))


def _edge_partition_body(src_hbm, dst_hbm, lsrc_hbm, ldst_hbm, counts_hbm,
                         schunk, dchunk, lsv, ldv, cbuf, ssem, dsem):
    wid = _wid()
    base = wid * CAPW
    lo = wid * RPW
    hi = lo + RPW
    zero16 = jnp.zeros((16,), jnp.int32)
    dummy16 = jnp.full((16,), RPW, jnp.int32)

    def init_body(i, c):
        lsv[pl.ds(i * 16, 16)] = zero16
        ldv[pl.ds(i * 16, 16)] = dummy16
        return c

    lax.fori_loop(0, VCAP // 16, init_body, 0)

    def flush(nflush):
        pltpu.sync_copy(lsv.at[pl.ds(0, FLUSH)],
                        lsrc_hbm.at[pl.ds(base + nflush * FLUSH, FLUSH)])
        pltpu.sync_copy(ldv.at[pl.ds(0, FLUSH)],
                        ldst_hbm.at[pl.ds(base + nflush * FLUSH, FLUSH)])

    def start_chunk(ci, buf):
        pltpu.async_copy(src_hbm.at[pl.ds(ci * CH, CH)],
                         schunk.at[buf], ssem.at[buf])
        pltpu.async_copy(dst_hbm.at[pl.ds(ci * CH, CH)],
                         dchunk.at[buf], dsem.at[buf])

    def wait_chunk(buf):
        pltpu.make_async_copy(src_hbm.at[pl.ds(0, CH)],
                              schunk.at[buf], ssem.at[buf]).wait()
        pltpu.make_async_copy(dst_hbm.at[pl.ds(0, CH)],
                              dchunk.at[buf], dsem.at[buf]).wait()

    start_chunk(0, 0)

    def chunk_body(ci, carry):
        buf = lax.rem(ci, 2)

        @pl.when(ci + 1 < E // CH)
        def _():
            start_chunk(ci + 1, 1 - buf)

        wait_chunk(buf)

        def blk8(g8, c2):
            cnt_buf, nflush = c2
            g = g8 * 8
            pms = []
            ss = []
            dls = []
            ms = []
            for k in range(8):
                d = dchunk[buf, pl.ds((g + k) * 16, 16)]
                s = schunk[buf, pl.ds((g + k) * 16, 16)]
                dl = d - lo
                m = dl.astype(jnp.uint32) < jnp.uint32(RPW)
                pms.append(plsc.cumsum(m.astype(jnp.int32)))
                ss.append(s)
                dls.append(dl)
                ms.append(m)
            c = cnt_buf
            for k in range(8):
                pos = c + pms[k] - 1
                plsc.store_scatter(lsv, [pos], ss[k], mask=ms[k])
                plsc.store_scatter(ldv, [pos], dls[k], mask=ms[k])
                c = c + pms[k][15]
            cnt_buf = c
            full = cnt_buf >= FLUSH

            @pl.when(full)
            def _():
                flush(nflush)
                for k in range(8):
                    t1 = lsv[pl.ds(FLUSH + k * 16, 16)]
                    lsv[pl.ds(k * 16, 16)] = t1
                    t2 = ldv[pl.ds(FLUSH + k * 16, 16)]
                    ldv[pl.ds(k * 16, 16)] = t2

            cnt_buf = jnp.where(full, cnt_buf - FLUSH, cnt_buf)
            nflush = jnp.where(full, nflush + 1, nflush)
            return (cnt_buf, nflush)

        return lax.fori_loop(0, CH // 128, blk8, carry)

    cnt_buf, nflush = lax.fori_loop(0, E // CH, chunk_body,
                                    (jnp.int32(0), jnp.int32(0)))

    @pl.when(cnt_buf > 0)
    def _():
        flush(nflush)

    total = nflush * FLUSH + cnt_buf
    cbuf[...] = jnp.full((16,), 1, jnp.int32) * total
    pltpu.sync_copy(cbuf, counts_hbm.at[pl.ds(wid * 16, 16)])


_edge_partition = functools.partial(
    pl.kernel,
    out_type=[
        jax.ShapeDtypeStruct((NW * CAPW,), jnp.int32),
        jax.ShapeDtypeStruct((NW * CAPW,), jnp.int32),
        jax.ShapeDtypeStruct((NW * 16,), jnp.int32),
    ],
    mesh=_mesh,
    compiler_params=_sc_params,
    scratch_types=[
        pltpu.VMEM((2, CH), jnp.int32),
        pltpu.VMEM((2, CH), jnp.int32),
        pltpu.VMEM((VCAP,), jnp.int32),
        pltpu.VMEM((VCAP,), jnp.int32),
        pltpu.VMEM((16,), jnp.int32),
        pltpu.SemaphoreType.DMA((2,)),
        pltpu.SemaphoreType.DMA((2,)),
    ],
)(_edge_partition_body)


def _segmin_body(b_hbm, lsrc_hbm, ldst_hbm, counts_hbm, c_hbm,
                 i0, i1, i2, d0, d1, d2, r0, r1, r2, cbuf, acc,
                 is0, is1, is2, ds0, ds1, ds2, rs0, rs1, rs2):
    wid = _wid()
    base = wid * CAPW
    infv = jnp.full((16,), jnp.inf, jnp.float32)
    idxb = (i0, i1, i2)
    dlocb = (d0, d1, d2)
    rows = (r0, r1, r2)
    isem = (is0, is1, is2)
    dsem = (ds0, ds1, ds2)
    rsem = (rs0, rs1, rs2)

    def init_body(r, c):
        acc[pl.ds(r * 16, 16)] = infv
        return c

    lax.fori_loop(0, (RPW + 1) * 8, init_body, 0)

    pltpu.sync_copy(counts_hbm.at[pl.ds(wid * 16, 16)], cbuf)
    cnt = cbuf[...][0]
    nblk = lax.div(cnt + (GB - 1), GB)

    def start_idx(b, s):
        pltpu.async_copy(lsrc_hbm.at[pl.ds(base + b * GB, GB)],
                         idxb[s], isem[s])
        pltpu.async_copy(ldst_hbm.at[pl.ds(base + b * GB, GB)],
                         dlocb[s], dsem[s])

    def wait_idx(s):
        pltpu.make_async_copy(lsrc_hbm.at[pl.ds(0, GB)],
                              idxb[s], isem[s]).wait()
        pltpu.make_async_copy(ldst_hbm.at[pl.ds(0, GB)],
                              dlocb[s], dsem[s]).wait()

    def start_rows(s):
        pltpu.async_copy(b_hbm.at[idxb[s]], rows[s], rsem[s])

    def wait_rows(s):
        pltpu.make_async_copy(b_hbm.at[pl.ds(0, GB)],
                              rows[s], rsem[s]).wait()

    def compute(s):
        rr = rows[s]
        dd = dlocb[s]

        def sg_body(sg, c2):
            a16 = dd[pl.ds(sg * 16, 16)] * HID
            for j in range(16):
                a = a16[j]
                e = sg * 16 + j
                bvs = [rr[e, pl.ds(v * 16, 16)] for v in range(8)]
                avs = [acc[pl.ds(a + v * 16, 16)] for v in range(8)]
                mns = [jnp.minimum(avs[v], bvs[v]) for v in range(8)]
                for v in range(8):
                    acc[pl.ds(a + v * 16, 16)] = mns[v]
            return c2

        lax.fori_loop(0, GB // 16, sg_body, 0)

    def do_block(b, s):
        s1 = (s + 1) % 3
        s2 = (s + 2) % 3

        @pl.when(b + 1 < nblk)
        def _():
            wait_idx(s1)
            start_rows(s1)

        wait_rows(s)

        @pl.when(b + 2 < nblk)
        def _():
            start_idx(b + 2, s2)

        compute(s)

    @pl.when(nblk > 0)
    def _():
        start_idx(0, 0)

    @pl.when(nblk > 1)
    def _():
        start_idx(1, 1)

    @pl.when(nblk > 0)
    def _():
        wait_idx(0)
        start_rows(0)

    def macro_body(g, c):
        b = g * 3
        for k in range(3):
            @pl.when(b + k < nblk)
            def _(k=k):
                do_block(b + k, k)
        return c

    lax.fori_loop(0, lax.div(nblk + 2, 3), macro_body, 0)
    pltpu.sync_copy(acc.at[pl.ds(0, RPW * HID)],
                    c_hbm.at[pl.ds(wid * RPW * HID, RPW * HID)])


_segmin = functools.partial(
    pl.kernel,
    out_type=jax.ShapeDtypeStruct((NPAD * HID,), jnp.float32),
    mesh=_mesh,
    compiler_params=_sc_params,
    scratch_types=(
        [pltpu.VMEM((GB,), jnp.int32)] * 6
        + [pltpu.VMEM((GB, HID), jnp.float32)] * 3
        + [pltpu.VMEM((16,), jnp.int32),
           pltpu.VMEM(((RPW + 1) * HID,), jnp.float32)]
        + [pltpu.SemaphoreType.DMA] * 9
    ),
)(_segmin_body)


def kernel(x, edge_index, Wt1, bt1, Wp1, bp1, Wt2, bt2, Wp2, bp2, Wt3, bt3,
           Wp3, bp3, Wl, bl, Wl2, bl2, Wo, bo):
    src = edge_index[0]
    dst = edge_index[1]
    lsrc, ldst, counts = _edge_partition(src, dst)
    ws = ((Wt1, bt1, Wp1, bp1), (Wt2, bt2, Wp2, bp2), (Wt3, bt3, Wp3, bp3))

    def wb(Wt, bt, Wp, bp):
        return (jnp.concatenate([Wt + Wp, Wt], axis=1),
                jnp.concatenate([bt + bp, jnp.zeros_like(bt)]))

    W, b = wb(*ws[0])
    A, B = _dense(x, W, b)
    for lyr in (1, 2):
        C = _segmin(B, lsrc, ldst, counts).reshape(NPAD, HID)[:N]
        W, b = wb(*ws[lyr])
        A, B = _dense_fused(A, C, W, b)
    C = _segmin(B, lsrc, ldst, counts).reshape(NPAD, HID)[:N]
    s = _head(A, C, Wl, bl, Wl2, bl2)
    out = (s / N) @ Wo + bo
    return out.reshape(-1)
